# baseline (speedup 1.0000x reference)
import math

import jax
import jax.numpy as jnp
from jax import lax
from jax.experimental import pallas as pl
from jax.experimental.pallas import tpu as pltpu

_BBOX_CLIP = float(math.log(1000.0 / 16.0))
_PRE_K = 1000
_NUM_PROPOSALS = 1000
_NMS_IOU = 0.7
_KP = 1024


def _decode_boxes(boxes, anchors):
    ay1 = anchors[..., 0]; ax1 = anchors[..., 1]; ay2 = anchors[..., 2]; ax2 = anchors[..., 3]
    ah = ay2 - ay1; aw = ax2 - ax1
    ayc = ay1 + 0.5 * ah; axc = ax1 + 0.5 * aw
    dy = boxes[..., 0]; dx = boxes[..., 1]
    dh = jnp.minimum(boxes[..., 2], _BBOX_CLIP)
    dw = jnp.minimum(boxes[..., 3], _BBOX_CLIP)
    nyc = dy * ah + ayc; nxc = dx * aw + axc
    nh = jnp.exp(dh) * ah; nw = jnp.exp(dw) * aw
    return jnp.stack([nyc - 0.5 * nh, nxc - 0.5 * nw, nyc + 0.5 * nh, nxc + 0.5 * nw], axis=-1)


def _clip_boxes(boxes, img):
    h = img[..., 0:1]; w = img[..., 1:2]
    maxes = jnp.concatenate([h, w, h, w], axis=-1)
    return jnp.clip(boxes, 0.0, maxes)


def _nms_keep_kernel(coords_smem, coords_ref, keep_ref):
    y1 = coords_ref[0, 0]
    x1 = coords_ref[0, 1]
    y2 = coords_ref[0, 2]
    x2 = coords_ref[0, 3]
    area = jnp.maximum(y2 - y1, 0.0) * jnp.maximum(x2 - x1, 0.0)
    flat = (lax.broadcasted_iota(jnp.int32, (8, 128), 0) * 128
            + lax.broadcasted_iota(jnp.int32, (8, 128), 1))

    def body(i, keep):
        oh = (flat == i).astype(jnp.float32)
        ki = jnp.sum(keep * oh)

        def do(keep):
            y1i = coords_smem[0, 0, i]
            x1i = coords_smem[0, 1, i]
            y2i = coords_smem[0, 2, i]
            x2i = coords_smem[0, 3, i]
            ai = jnp.maximum(y2i - y1i, 0.0) * jnp.maximum(x2i - x1i, 0.0)
            ih = jnp.maximum(jnp.minimum(y2, y2i) - jnp.maximum(y1, y1i), 0.0)
            iw = jnp.maximum(jnp.minimum(x2, x2i) - jnp.maximum(x1, x1i), 0.0)
            inter = ih * iw
            union = ai + area - inter
            iou = inter / (union + 1e-8)
            supp = jnp.logical_and(iou > _NMS_IOU, flat > i).astype(jnp.float32)
            return keep * (1.0 - supp)

        return lax.cond(ki > 0.5, do, lambda keep: keep, keep)

    keep = lax.fori_loop(0, _KP, body, jnp.ones((8, 128), jnp.float32))
    keep_ref[0] = keep


def _nms_keep(coords_flat, coords):
    return pl.pallas_call(
        _nms_keep_kernel,
        grid=(coords.shape[0],),
        in_specs=[
            pl.BlockSpec((1, 4, _KP), lambda i: (i, 0, 0),
                         memory_space=pltpu.SMEM),
            pl.BlockSpec((1, 4, 8, 128), lambda i: (i, 0, 0, 0)),
        ],
        out_specs=pl.BlockSpec((1, 8, 128), lambda i: (i, 0, 0)),
        out_shape=jax.ShapeDtypeStruct((coords.shape[0], 8, 128), jnp.float32),
    )(coords_flat, coords)


def kernel(raw_boxes_l3, raw_scores_l3, anchor_boxes_l3,
           raw_boxes_l4, raw_scores_l4, anchor_boxes_l4,
           raw_boxes_l5, raw_scores_l5, anchor_boxes_l5,
           image_shape):
    levels = [
        (raw_boxes_l3, raw_scores_l3, anchor_boxes_l3),
        (raw_boxes_l4, raw_scores_l4, anchor_boxes_l4),
        (raw_boxes_l5, raw_scores_l5, anchor_boxes_l5),
    ]
    img = image_shape[:, None, :]
    per_level = []
    padded = []
    for rb, rs, ab in levels:
        B, fh, fw, na = rs.shape
        nb = fh * fw * na
        sc = jax.nn.sigmoid(jnp.reshape(rs, (B, nb)))
        bx = jnp.reshape(rb, (B, nb, 4))
        an = jnp.reshape(ab, (B, nb, 4)).astype(sc.dtype)
        pre_k = min(nb, _PRE_K)
        sck, idx = lax.top_k(sc, pre_k)
        bxk = jnp.take_along_axis(bx, idx[..., None], axis=1)
        ank = jnp.take_along_axis(an, idx[..., None], axis=1)
        dec = _clip_boxes(_decode_boxes(bxk, ank), img)
        pad = _KP - pre_k
        padded.append(jnp.pad(dec, ((0, 0), (0, pad), (0, 0))))
        per_level.append((sck, dec, pre_k))

    boxes_all = jnp.concatenate(padded, axis=0)
    coords_flat = jnp.transpose(boxes_all, (0, 2, 1))
    coords = coords_flat.reshape(boxes_all.shape[0], 4, 8, 128)
    keep_all = _nms_keep(coords_flat, coords).reshape(boxes_all.shape[0], _KP)

    B = levels[0][1].shape[0]
    rois_list, score_list = [], []
    for li, (sck, dec, pre_k) in enumerate(per_level):
        kp = keep_all[li * B:(li + 1) * B, :pre_k] > 0.5
        masked = jnp.where(kp, lax.stop_gradient(sck), -1.0)
        sel_masked, idx = lax.top_k(masked, pre_k)
        valid = sel_masked > -0.5
        ssc = jnp.where(valid, jnp.take_along_axis(sck, idx, axis=1), 0.0)
        sbx = jnp.where(valid[..., None], jnp.take_along_axis(dec, idx[..., None], axis=1), 0.0)
        rois_list.append(sbx)
        score_list.append(ssc)

    all_rois = jnp.concatenate(rois_list, axis=1)
    all_scores = jnp.concatenate(score_list, axis=1)
    k = min(all_scores.shape[1], _NUM_PROPOSALS)
    fsc, fidx = lax.top_k(all_scores, k)
    frois = jnp.take_along_axis(all_rois, fidx[..., None], axis=1)
    return (frois, fsc)


# speedup vs baseline: 1.1637x; 1.1637x over previous
import math

import jax
import jax.numpy as jnp
from jax import lax
from jax.experimental import pallas as pl
from jax.experimental.pallas import tpu as pltpu

_BBOX_CLIP = float(math.log(1000.0 / 16.0))
_PRE_K = 1000
_NUM_PROPOSALS = 1000
_NMS_IOU = 0.7
_KP = 1024


def _decode_boxes(boxes, anchors):
    ay1 = anchors[..., 0]; ax1 = anchors[..., 1]; ay2 = anchors[..., 2]; ax2 = anchors[..., 3]
    ah = ay2 - ay1; aw = ax2 - ax1
    ayc = ay1 + 0.5 * ah; axc = ax1 + 0.5 * aw
    dy = boxes[..., 0]; dx = boxes[..., 1]
    dh = jnp.minimum(boxes[..., 2], _BBOX_CLIP)
    dw = jnp.minimum(boxes[..., 3], _BBOX_CLIP)
    nyc = dy * ah + ayc; nxc = dx * aw + axc
    nh = jnp.exp(dh) * ah; nw = jnp.exp(dw) * aw
    return jnp.stack([nyc - 0.5 * nh, nxc - 0.5 * nw, nyc + 0.5 * nh, nxc + 0.5 * nw], axis=-1)


def _clip_boxes(boxes, img):
    h = img[..., 0:1]; w = img[..., 1:2]
    maxes = jnp.concatenate([h, w, h, w], axis=-1)
    return jnp.clip(boxes, 0.0, maxes)


def _nms_keep_kernel(coords_smem, coords_ref, keep_ref):
    y1 = coords_ref[0, 0]
    x1 = coords_ref[0, 1]
    y2 = coords_ref[0, 2]
    x2 = coords_ref[0, 3]
    area = jnp.maximum(y2 - y1, 0.0) * jnp.maximum(x2 - x1, 0.0)
    flat = (lax.broadcasted_iota(jnp.int32, (8, 128), 0) * 128
            + lax.broadcasted_iota(jnp.int32, (8, 128), 1))

    def body(i, keep):
        oh = (flat == i).astype(jnp.float32)
        ki = jnp.sum(keep * oh)
        y1i = coords_smem[0, 0, i]
        x1i = coords_smem[0, 1, i]
        y2i = coords_smem[0, 2, i]
        x2i = coords_smem[0, 3, i]
        ai = jnp.maximum(y2i - y1i, 0.0) * jnp.maximum(x2i - x1i, 0.0)
        ih = jnp.maximum(jnp.minimum(y2, y2i) - jnp.maximum(y1, y1i), 0.0)
        iw = jnp.maximum(jnp.minimum(x2, x2i) - jnp.maximum(x1, x1i), 0.0)
        inter = ih * iw
        union = ai + area - inter
        iou = inter / (union + 1e-8)
        supp = jnp.logical_and(iou > _NMS_IOU, flat > i).astype(jnp.float32)
        return keep * (1.0 - ki * supp)

    keep = lax.fori_loop(0, _KP, body, jnp.ones((8, 128), jnp.float32))
    keep_ref[0] = keep


def _nms_keep(coords_flat, coords):
    return pl.pallas_call(
        _nms_keep_kernel,
        grid=(coords.shape[0],),
        in_specs=[
            pl.BlockSpec((1, 4, _KP), lambda i: (i, 0, 0),
                         memory_space=pltpu.SMEM),
            pl.BlockSpec((1, 4, 8, 128), lambda i: (i, 0, 0, 0)),
        ],
        out_specs=pl.BlockSpec((1, 8, 128), lambda i: (i, 0, 0)),
        out_shape=jax.ShapeDtypeStruct((coords.shape[0], 8, 128), jnp.float32),
    )(coords_flat, coords)


def kernel(raw_boxes_l3, raw_scores_l3, anchor_boxes_l3,
           raw_boxes_l4, raw_scores_l4, anchor_boxes_l4,
           raw_boxes_l5, raw_scores_l5, anchor_boxes_l5,
           image_shape):
    levels = [
        (raw_boxes_l3, raw_scores_l3, anchor_boxes_l3),
        (raw_boxes_l4, raw_scores_l4, anchor_boxes_l4),
        (raw_boxes_l5, raw_scores_l5, anchor_boxes_l5),
    ]
    img = image_shape[:, None, :]
    per_level = []
    padded = []
    for rb, rs, ab in levels:
        B, fh, fw, na = rs.shape
        nb = fh * fw * na
        sc = jax.nn.sigmoid(jnp.reshape(rs, (B, nb)))
        bx = jnp.reshape(rb, (B, nb, 4))
        an = jnp.reshape(ab, (B, nb, 4)).astype(sc.dtype)
        pre_k = min(nb, _PRE_K)
        sck, idx = lax.top_k(sc, pre_k)
        bxk = jnp.take_along_axis(bx, idx[..., None], axis=1)
        ank = jnp.take_along_axis(an, idx[..., None], axis=1)
        dec = _clip_boxes(_decode_boxes(bxk, ank), img)
        pad = _KP - pre_k
        padded.append(jnp.pad(dec, ((0, 0), (0, pad), (0, 0))))
        per_level.append((sck, dec, pre_k))

    boxes_all = jnp.concatenate(padded, axis=0)
    coords_flat = jnp.transpose(boxes_all, (0, 2, 1))
    coords = coords_flat.reshape(boxes_all.shape[0], 4, 8, 128)
    keep_all = _nms_keep(coords_flat, coords).reshape(boxes_all.shape[0], _KP)

    B = levels[0][1].shape[0]
    rois_list, score_list = [], []
    for li, (sck, dec, pre_k) in enumerate(per_level):
        kp = keep_all[li * B:(li + 1) * B, :pre_k] > 0.5
        masked = jnp.where(kp, lax.stop_gradient(sck), -1.0)
        sel_masked, idx = lax.top_k(masked, pre_k)
        valid = sel_masked > -0.5
        ssc = jnp.where(valid, jnp.take_along_axis(sck, idx, axis=1), 0.0)
        sbx = jnp.where(valid[..., None], jnp.take_along_axis(dec, idx[..., None], axis=1), 0.0)
        rois_list.append(sbx)
        score_list.append(ssc)

    all_rois = jnp.concatenate(rois_list, axis=1)
    all_scores = jnp.concatenate(score_list, axis=1)
    k = min(all_scores.shape[1], _NUM_PROPOSALS)
    fsc, fidx = lax.top_k(all_scores, k)
    frois = jnp.take_along_axis(all_rois, fidx[..., None], axis=1)
    return (frois, fsc)


# speedup vs baseline: 1.1637x; 1.0001x over previous
import math

import jax
import jax.numpy as jnp
from jax import lax
from jax.experimental import pallas as pl
from jax.experimental.pallas import tpu as pltpu

_BBOX_CLIP = float(math.log(1000.0 / 16.0))
_PRE_K = 1000
_NUM_PROPOSALS = 1000
_NMS_IOU = 0.7
_KP = 1024


def _decode_boxes(boxes, anchors):
    ay1 = anchors[..., 0]; ax1 = anchors[..., 1]; ay2 = anchors[..., 2]; ax2 = anchors[..., 3]
    ah = ay2 - ay1; aw = ax2 - ax1
    ayc = ay1 + 0.5 * ah; axc = ax1 + 0.5 * aw
    dy = boxes[..., 0]; dx = boxes[..., 1]
    dh = jnp.minimum(boxes[..., 2], _BBOX_CLIP)
    dw = jnp.minimum(boxes[..., 3], _BBOX_CLIP)
    nyc = dy * ah + ayc; nxc = dx * aw + axc
    nh = jnp.exp(dh) * ah; nw = jnp.exp(dw) * aw
    return jnp.stack([nyc - 0.5 * nh, nxc - 0.5 * nw, nyc + 0.5 * nh, nxc + 0.5 * nw], axis=-1)


def _clip_boxes(boxes, img):
    h = img[..., 0:1]; w = img[..., 1:2]
    maxes = jnp.concatenate([h, w, h, w], axis=-1)
    return jnp.clip(boxes, 0.0, maxes)


def _nms_keep_kernel(coords_smem, coords_ref, keep_ref):
    y1 = coords_ref[0, 0]
    x1 = coords_ref[0, 1]
    y2 = coords_ref[0, 2]
    x2 = coords_ref[0, 3]
    area = jnp.maximum(y2 - y1, 0.0) * jnp.maximum(x2 - x1, 0.0)
    flat = (lax.broadcasted_iota(jnp.int32, (8, 128), 0) * 128
            + lax.broadcasted_iota(jnp.int32, (8, 128), 1))

    def body(i, keep):
        oh = (flat == i).astype(jnp.float32)
        ki = jnp.sum(keep * oh)
        y1i = coords_smem[0, 0, i]
        x1i = coords_smem[0, 1, i]
        y2i = coords_smem[0, 2, i]
        x2i = coords_smem[0, 3, i]
        ai = jnp.maximum(y2i - y1i, 0.0) * jnp.maximum(x2i - x1i, 0.0)
        ih = jnp.maximum(jnp.minimum(y2, y2i) - jnp.maximum(y1, y1i), 0.0)
        iw = jnp.maximum(jnp.minimum(x2, x2i) - jnp.maximum(x1, x1i), 0.0)
        inter = ih * iw
        union = ai + area - inter
        iou = inter / (union + 1e-8)
        supp = jnp.logical_and(iou > _NMS_IOU, flat > i).astype(jnp.float32)
        return keep * (1.0 - ki * supp)

    keep = lax.fori_loop(0, _KP, body, jnp.ones((8, 128), jnp.float32))
    keep_ref[0] = keep


def _nms_keep(coords_flat, coords):
    return pl.pallas_call(
        _nms_keep_kernel,
        grid=(coords.shape[0],),
        in_specs=[
            pl.BlockSpec((1, 4, _KP), lambda i: (i, 0, 0),
                         memory_space=pltpu.SMEM),
            pl.BlockSpec((1, 4, 8, 128), lambda i: (i, 0, 0, 0)),
        ],
        out_specs=pl.BlockSpec((1, 8, 128), lambda i: (i, 0, 0)),
        out_shape=jax.ShapeDtypeStruct((coords.shape[0], 8, 128), jnp.float32),
        compiler_params=pltpu.CompilerParams(dimension_semantics=("parallel",)),
    )(coords_flat, coords)


def kernel(raw_boxes_l3, raw_scores_l3, anchor_boxes_l3,
           raw_boxes_l4, raw_scores_l4, anchor_boxes_l4,
           raw_boxes_l5, raw_scores_l5, anchor_boxes_l5,
           image_shape):
    levels = [
        (raw_boxes_l3, raw_scores_l3, anchor_boxes_l3),
        (raw_boxes_l4, raw_scores_l4, anchor_boxes_l4),
        (raw_boxes_l5, raw_scores_l5, anchor_boxes_l5),
    ]
    img = image_shape[:, None, :]
    per_level = []
    padded = []
    for rb, rs, ab in levels:
        B, fh, fw, na = rs.shape
        nb = fh * fw * na
        sc = jax.nn.sigmoid(jnp.reshape(rs, (B, nb)))
        bx = jnp.reshape(rb, (B, nb, 4))
        an = jnp.reshape(ab, (B, nb, 4)).astype(sc.dtype)
        pre_k = min(nb, _PRE_K)
        sck, idx = lax.top_k(sc, pre_k)
        bxk = jnp.take_along_axis(bx, idx[..., None], axis=1)
        ank = jnp.take_along_axis(an, idx[..., None], axis=1)
        dec = _clip_boxes(_decode_boxes(bxk, ank), img)
        pad = _KP - pre_k
        padded.append(jnp.pad(dec, ((0, 0), (0, pad), (0, 0))))
        per_level.append((sck, dec, pre_k))

    boxes_all = jnp.concatenate(padded, axis=0)
    coords_flat = jnp.transpose(boxes_all, (0, 2, 1))
    coords = coords_flat.reshape(boxes_all.shape[0], 4, 8, 128)
    keep_all = _nms_keep(coords_flat, coords).reshape(boxes_all.shape[0], _KP)

    B = levels[0][1].shape[0]
    rois_list, score_list = [], []
    for li, (sck, dec, pre_k) in enumerate(per_level):
        kp = keep_all[li * B:(li + 1) * B, :pre_k] > 0.5
        masked = jnp.where(kp, lax.stop_gradient(sck), -1.0)
        sel_masked, idx = lax.top_k(masked, pre_k)
        valid = sel_masked > -0.5
        ssc = jnp.where(valid, jnp.take_along_axis(sck, idx, axis=1), 0.0)
        sbx = jnp.where(valid[..., None], jnp.take_along_axis(dec, idx[..., None], axis=1), 0.0)
        rois_list.append(sbx)
        score_list.append(ssc)

    all_rois = jnp.concatenate(rois_list, axis=1)
    all_scores = jnp.concatenate(score_list, axis=1)
    k = min(all_scores.shape[1], _NUM_PROPOSALS)
    fsc, fidx = lax.top_k(all_scores, k)
    frois = jnp.take_along_axis(all_rois, fidx[..., None], axis=1)
    return (frois, fsc)
